# Initial kernel scaffold; baseline (speedup 1.0000x reference)
import jax
import jax.numpy as jnp
from jax.experimental import pallas as pl


def kernel(x, table):
    raise NotImplementedError("write your pallas kernel here")


# speedup vs baseline: 1.6830x; 1.6830x over previous
import functools

import jax
import jax.numpy as jnp
from jax import lax
from jax.experimental import pallas as pl
from jax.experimental.pallas import tpu as pltpu
from jax.experimental.pallas import tpu_sc as plsc

_HIDDEN = 64
_CHUNK = 512
_SUB = 128
_LANES = 16


@functools.lru_cache(maxsize=None)
def _make_kernel(n_rows):
    info = plsc.get_sparse_core_info()
    nc, ns = info.num_cores, info.num_subcores
    nw = nc * ns
    per_w = n_rows // nw
    n_chunks = per_w // _CHUNK
    assert per_w * nw == n_rows and n_chunks * _CHUNK == per_w

    mesh = plsc.VectorSubcoreMesh(core_axis_name="c", subcore_axis_name="s")

    @functools.partial(
        pl.kernel,
        mesh=mesh,
        compiler_params=pltpu.CompilerParams(
            needs_layout_passes=False, use_tc_tiling_on_sc=False
        ),
        out_type=jax.ShapeDtypeStruct((n_rows, _HIDDEN), jnp.float32),
        scratch_types=[
            pltpu.VMEM((2 * _CHUNK,), jnp.float32),
            pltpu.VMEM((_CHUNK,), jnp.int32),
            pltpu.VMEM((_CHUNK,), jnp.float32),
            pltpu.VMEM((_CHUNK, _HIDDEN), jnp.float32),
            pltpu.SemaphoreType.DMA,
        ],
    )
    def k(x_hbm, table_hbm, out_hbm, xbuf, idxbuf, durbuf, rowsbuf, sem):
        wid = lax.axis_index("s") * nc + lax.axis_index("c")
        base = wid * per_w
        iota = lax.iota(jnp.int32, _LANES)

        def chunk_body(g, carry):
            off = base + g * _CHUNK
            pltpu.sync_copy(x_hbm.at[pl.ds(2 * off, 2 * _CHUNK)], xbuf)

            def extract(j, carry2):
                r2 = 2 * (j * _LANES + iota)
                idxf = plsc.load_gather(xbuf, [r2])
                durf = plsc.load_gather(xbuf, [r2 + 1])
                idxbuf[pl.ds(j * _LANES, _LANES)] = idxf.astype(jnp.int32)
                durbuf[pl.ds(j * _LANES, _LANES)] = durf
                return carry2

            lax.fori_loop(0, _CHUNK // _LANES, extract, 0)

            copies = []
            for s in range(_CHUNK // _SUB):
                copies.append(
                    pltpu.async_copy(
                        table_hbm.at[idxbuf.at[pl.ds(s * _SUB, _SUB)]],
                        rowsbuf.at[pl.ds(s * _SUB, _SUB), :],
                        sem,
                    )
                )
            for c in copies:
                c.wait()

            c63 = jnp.full((_LANES,), _HIDDEN - 1, jnp.int32)

            def durscatter(j, carry2):
                r = j * _LANES + iota
                durv = durbuf[pl.ds(j * _LANES, _LANES)]
                plsc.store_scatter(rowsbuf, [r, c63], durv)
                return carry2

            lax.fori_loop(0, _CHUNK // _LANES, durscatter, 0)

            pltpu.sync_copy(rowsbuf, out_hbm.at[pl.ds(off, _CHUNK), :])
            return carry

        lax.fori_loop(0, n_chunks, chunk_body, 0)

    return k


def kernel(x, table):
    b, l, _ = x.shape
    n = b * l
    xf = x.reshape(2 * n)
    table_pad = jnp.pad(table, ((0, 0), (0, _HIDDEN - table.shape[1])))
    out = _make_kernel(n)(xf, table_pad)
    return out.reshape(b, l, _HIDDEN)


# speedup vs baseline: 1.7552x; 1.0429x over previous
import functools

import jax
import jax.numpy as jnp
from jax import lax
from jax.experimental import pallas as pl
from jax.experimental.pallas import tpu as pltpu
from jax.experimental.pallas import tpu_sc as plsc

_HIDDEN = 64
_CHUNK = 512
_SUB = 128
_LANES = 16


@functools.lru_cache(maxsize=None)
def _make_kernel(n_rows):
    info = plsc.get_sparse_core_info()
    nc, ns = info.num_cores, info.num_subcores
    nw = nc * ns
    per_w = n_rows // nw
    n_chunks = per_w // _CHUNK
    assert per_w * nw == n_rows and n_chunks * _CHUNK == per_w
    assert n_chunks % 2 == 0 and n_chunks >= 4

    mesh = plsc.VectorSubcoreMesh(core_axis_name="c", subcore_axis_name="s")

    @functools.partial(
        pl.kernel,
        mesh=mesh,
        compiler_params=pltpu.CompilerParams(
            needs_layout_passes=False, use_tc_tiling_on_sc=False
        ),
        out_type=jax.ShapeDtypeStruct((n_rows, _HIDDEN), jnp.float32),
        scratch_types=[
            pltpu.VMEM((2 * 2 * _CHUNK,), jnp.float32),
            pltpu.VMEM((2 * _CHUNK,), jnp.int32),
            pltpu.VMEM((2 * _CHUNK,), jnp.float32),
            pltpu.VMEM((2 * _CHUNK, _HIDDEN), jnp.float32),
            pltpu.SemaphoreType.DMA,
            pltpu.SemaphoreType.DMA,
            pltpu.SemaphoreType.DMA,
            pltpu.SemaphoreType.DMA,
            pltpu.SemaphoreType.DMA,
            pltpu.SemaphoreType.DMA,
        ],
    )
    def k(x_hbm, table_hbm, out_hbm, xbuf, idxbuf, durbuf, rowsbuf,
          sin0, sin1, sg0, sg1, sout0, sout1):
        sin = (sin0, sin1)
        sg = (sg0, sg1)
        sout = (sout0, sout1)
        wid = lax.axis_index("s") * nc + lax.axis_index("c")
        base = wid * per_w
        iota = lax.iota(jnp.int32, _LANES)
        c63 = jnp.full((_LANES,), _HIDDEN - 1, jnp.int32)

        def in_copy(g, p):
            return pltpu.make_async_copy(
                x_hbm.at[pl.ds(2 * (base + g * _CHUNK), 2 * _CHUNK)],
                xbuf.at[pl.ds(p * 2 * _CHUNK, 2 * _CHUNK)],
                sin[p],
            )

        def extract(p):
            for j in range(_CHUNK // _LANES):
                r2 = p * 2 * _CHUNK + 2 * (j * _LANES) + 2 * iota
                idxf = plsc.load_gather(xbuf, [r2])
                durf = plsc.load_gather(xbuf, [r2 + 1])
                idxbuf[pl.ds(p * _CHUNK + j * _LANES, _LANES)] = (
                    idxf.astype(jnp.int32))
                durbuf[pl.ds(p * _CHUNK + j * _LANES, _LANES)] = durf

        def fire_gathers(p):
            for s in range(_CHUNK // _SUB):
                pltpu.async_copy(
                    table_hbm.at[idxbuf.at[pl.ds(p * _CHUNK + s * _SUB, _SUB)]],
                    rowsbuf.at[pl.ds(p * _CHUNK + s * _SUB, _SUB), :],
                    sg[p],
                )

        def wait_gathers(p):
            pltpu.make_async_copy(
                table_hbm.at[idxbuf.at[pl.ds(p * _CHUNK, _CHUNK)]],
                rowsbuf.at[pl.ds(p * _CHUNK, _CHUNK), :],
                sg[p],
            ).wait()

        def durscatter(p):
            for j in range(_CHUNK // _LANES):
                r = p * _CHUNK + j * _LANES + iota
                durv = durbuf[pl.ds(p * _CHUNK + j * _LANES, _LANES)]
                plsc.store_scatter(rowsbuf, [r, c63], durv)

        def out_copy(g, p):
            return pltpu.make_async_copy(
                rowsbuf.at[pl.ds(p * _CHUNK, _CHUNK), :],
                out_hbm.at[pl.ds(base + g * _CHUNK, _CHUNK), :],
                sout[p],
            )

        in_copy(0, 0).start()
        in_copy(0, 0).wait()
        extract(0)
        fire_gathers(0)
        in_copy(1, 1).start()

        def sub_body(g, p):
            pn = 1 - p

            @pl.when(g + 1 < n_chunks)
            def _():
                in_copy(g + 1, pn).wait()
                extract(pn)

            @pl.when(g + 2 < n_chunks)
            def _():
                in_copy(g + 2, p).start()

            wait_gathers(p)
            durscatter(p)

            @pl.when(g >= 1)
            def _():
                out_copy(g - 1, pn).wait()

            @pl.when(g + 1 < n_chunks)
            def _():
                fire_gathers(pn)

            out_copy(g, p).start()

        def macro(t, carry):
            sub_body(2 * t, 0)
            sub_body(2 * t + 1, 1)
            return carry

        lax.fori_loop(0, n_chunks // 2, macro, 0)
        out_copy(n_chunks - 1, 1).wait()

    return k


def kernel(x, table):
    b, l, _ = x.shape
    n = b * l
    xf = x.reshape(2 * n)
    table_pad = jnp.pad(table, ((0, 0), (0, _HIDDEN - table.shape[1])))
    out = _make_kernel(n)(xf, table_pad)
    return out.reshape(b, l, _HIDDEN)


# speedup vs baseline: 4.0629x; 2.3148x over previous
import functools

import jax
import jax.numpy as jnp
from jax import lax
from jax.experimental import pallas as pl
from jax.experimental.pallas import tpu as pltpu
from jax.experimental.pallas import tpu_sc as plsc

_HIDDEN = 64
_CHUNK = 512
_SUB = 128
_LANES = 16


@functools.lru_cache(maxsize=None)
def _make_kernel(n_rows):
    info = plsc.get_sparse_core_info()
    nc, ns = info.num_cores, info.num_subcores
    nw = nc * ns
    per_w = n_rows // nw
    n_chunks = per_w // _CHUNK
    assert per_w * nw == n_rows and n_chunks * _CHUNK == per_w
    assert n_chunks % 2 == 0 and n_chunks >= 4

    mesh = plsc.VectorSubcoreMesh(core_axis_name="c", subcore_axis_name="s")

    @functools.partial(
        pl.kernel,
        mesh=mesh,
        compiler_params=pltpu.CompilerParams(
            needs_layout_passes=False, use_tc_tiling_on_sc=False
        ),
        out_type=jax.ShapeDtypeStruct((n_rows, _HIDDEN), jnp.float32),
        scratch_types=[
            pltpu.VMEM((2 * _CHUNK,), jnp.int32),
            pltpu.VMEM((2 * _CHUNK,), jnp.float32),
            pltpu.VMEM((2 * _CHUNK, _HIDDEN), jnp.float32),
            pltpu.SemaphoreType.DMA,
            pltpu.SemaphoreType.DMA,
            pltpu.SemaphoreType.DMA,
            pltpu.SemaphoreType.DMA,
            pltpu.SemaphoreType.DMA,
            pltpu.SemaphoreType.DMA,
        ],
    )
    def k(idx_hbm, dur_hbm, table_hbm, out_hbm, idxbuf, durbuf, rowsbuf,
          sin0, sin1, sg0, sg1, sout0, sout1):
        sin = (sin0, sin1)
        sg = (sg0, sg1)
        sout = (sout0, sout1)
        wid = lax.axis_index("s") * nc + lax.axis_index("c")
        base = wid * per_w
        iota = lax.iota(jnp.int32, _LANES)
        c63 = jnp.full((_LANES,), _HIDDEN - 1, jnp.int32)

        def in_idx(g, p):
            return pltpu.make_async_copy(
                idx_hbm.at[pl.ds(base + g * _CHUNK, _CHUNK)],
                idxbuf.at[pl.ds(p * _CHUNK, _CHUNK)],
                sin[p],
            )

        def in_dur(g, p):
            return pltpu.make_async_copy(
                dur_hbm.at[pl.ds(base + g * _CHUNK, _CHUNK)],
                durbuf.at[pl.ds(p * _CHUNK, _CHUNK)],
                sin[p],
            )

        def fire_gathers(p):
            for s in range(_CHUNK // _SUB):
                pltpu.async_copy(
                    table_hbm.at[idxbuf.at[pl.ds(p * _CHUNK + s * _SUB, _SUB)]],
                    rowsbuf.at[pl.ds(p * _CHUNK + s * _SUB, _SUB), :],
                    sg[p],
                )

        def wait_gathers(p):
            pltpu.make_async_copy(
                table_hbm.at[idxbuf.at[pl.ds(p * _CHUNK, _CHUNK)]],
                rowsbuf.at[pl.ds(p * _CHUNK, _CHUNK), :],
                sg[p],
            ).wait()

        def durscatter(p):
            for j in range(_CHUNK // _LANES):
                r = p * _CHUNK + j * _LANES + iota
                durv = durbuf[pl.ds(p * _CHUNK + j * _LANES, _LANES)]
                plsc.store_scatter(rowsbuf, [r, c63], durv)

        def out_copy(g, p):
            return pltpu.make_async_copy(
                rowsbuf.at[pl.ds(p * _CHUNK, _CHUNK), :],
                out_hbm.at[pl.ds(base + g * _CHUNK, _CHUNK), :],
                sout[p],
            )

        in_idx(0, 0).start()
        in_dur(0, 0).start()
        in_idx(0, 0).wait()
        in_dur(0, 0).wait()
        fire_gathers(0)
        in_idx(1, 1).start()
        in_dur(1, 1).start()

        def sub_body(g, p):
            pn = 1 - p

            @pl.when(g + 1 < n_chunks)
            def _():
                in_idx(g + 1, pn).wait()
                in_dur(g + 1, pn).wait()

            wait_gathers(p)
            durscatter(p)

            @pl.when(g + 2 < n_chunks)
            def _():
                in_idx(g + 2, p).start()
                in_dur(g + 2, p).start()

            @pl.when(g >= 1)
            def _():
                out_copy(g - 1, pn).wait()

            @pl.when(g + 1 < n_chunks)
            def _():
                fire_gathers(pn)

            out_copy(g, p).start()

        def macro(t, carry):
            sub_body(2 * t, 0)
            sub_body(2 * t + 1, 1)
            return carry

        lax.fori_loop(0, n_chunks // 2, macro, 0)
        out_copy(n_chunks - 1, 1).wait()

    return k


def kernel(x, table):
    b, l, _ = x.shape
    n = b * l
    idx = x[..., 0].astype(jnp.int32).reshape(n)
    dur = x[..., 1].reshape(n)
    table_pad = jnp.pad(table, ((0, 0), (0, _HIDDEN - table.shape[1])))
    out = _make_kernel(n)(idx, dur, table_pad)
    return out.reshape(b, l, _HIDDEN)


# speedup vs baseline: 4.0634x; 1.0001x over previous
import functools

import jax
import jax.numpy as jnp
from jax import lax
from jax.experimental import pallas as pl
from jax.experimental.pallas import tpu as pltpu
from jax.experimental.pallas import tpu_sc as plsc

_HIDDEN = 64
_RPC = 4
_LANES = 16
_SUB = 128


@functools.lru_cache(maxsize=None)
def _make_kernel(b, l):
    n_rows = b * l
    chunk = _RPC * l
    info = plsc.get_sparse_core_info()
    nc, ns = info.num_cores, info.num_subcores
    nw = nc * ns
    per_w = n_rows // nw
    rows_w = b // nw
    n_chunks = per_w // chunk
    assert per_w * nw == n_rows and n_chunks * chunk == per_w
    assert n_chunks % 2 == 0 and n_chunks >= 4
    subs = []
    off = 0
    while off < chunk:
        sz = min(_SUB, chunk - off)
        subs.append((off, sz))
        off += sz

    mesh = plsc.VectorSubcoreMesh(core_axis_name="c", subcore_axis_name="s")

    @functools.partial(
        pl.kernel,
        mesh=mesh,
        compiler_params=pltpu.CompilerParams(
            needs_layout_passes=False, use_tc_tiling_on_sc=False
        ),
        out_type=jax.ShapeDtypeStruct((b, l, _HIDDEN), jnp.float32),
        scratch_types=[
            pltpu.VMEM((2 * chunk,), jnp.int32),
            pltpu.VMEM((2 * chunk,), jnp.float32),
            pltpu.VMEM((2 * chunk, _HIDDEN), jnp.float32),
            pltpu.SemaphoreType.DMA,
            pltpu.SemaphoreType.DMA,
            pltpu.SemaphoreType.DMA,
            pltpu.SemaphoreType.DMA,
            pltpu.SemaphoreType.DMA,
            pltpu.SemaphoreType.DMA,
        ],
    )
    def k(idx_hbm, dur_hbm, table_hbm, out_hbm, idxbuf, durbuf, rowsbuf,
          sin0, sin1, sg0, sg1, sout0, sout1):
        sin = (sin0, sin1)
        sg = (sg0, sg1)
        sout = (sout0, sout1)
        wid = lax.axis_index("s") * nc + lax.axis_index("c")
        base = wid * per_w
        rbase = wid * rows_w
        iota = lax.iota(jnp.int32, _LANES)
        c63 = jnp.full((_LANES,), _HIDDEN - 1, jnp.int32)

        def in_idx(g, p):
            return pltpu.make_async_copy(
                idx_hbm.at[pl.ds(base + g * chunk, chunk)],
                idxbuf.at[pl.ds(p * chunk, chunk)],
                sin[p],
            )

        def in_dur(g, p):
            return pltpu.make_async_copy(
                dur_hbm.at[pl.ds(base + g * chunk, chunk)],
                durbuf.at[pl.ds(p * chunk, chunk)],
                sin[p],
            )

        def fire_gathers(p):
            for s_off, s_sz in subs:
                pltpu.async_copy(
                    table_hbm.at[idxbuf.at[pl.ds(p * chunk + s_off, s_sz)]],
                    rowsbuf.at[pl.ds(p * chunk + s_off, s_sz), :],
                    sg[p],
                )

        def wait_gathers(p):
            pltpu.make_async_copy(
                table_hbm.at[idxbuf.at[pl.ds(p * chunk, chunk)]],
                rowsbuf.at[pl.ds(p * chunk, chunk), :],
                sg[p],
            ).wait()

        def durscatter(p):
            for j in range(chunk // _LANES):
                r = p * chunk + j * _LANES + iota
                durv = durbuf[pl.ds(p * chunk + j * _LANES, _LANES)]
                plsc.store_scatter(rowsbuf, [r, c63], durv)

        def out_copies(g, p):
            return [
                pltpu.make_async_copy(
                    rowsbuf.at[pl.ds(p * chunk + r * l, l), :],
                    out_hbm.at[rbase + g * _RPC + r],
                    sout[p],
                )
                for r in range(_RPC)
            ]

        in_idx(0, 0).start()
        in_dur(0, 0).start()
        in_idx(0, 0).wait()
        in_dur(0, 0).wait()
        fire_gathers(0)
        in_idx(1, 1).start()
        in_dur(1, 1).start()

        def sub_body(g, p):
            pn = 1 - p

            @pl.when(g + 1 < n_chunks)
            def _():
                in_idx(g + 1, pn).wait()
                in_dur(g + 1, pn).wait()

            wait_gathers(p)
            durscatter(p)

            @pl.when(g + 2 < n_chunks)
            def _():
                in_idx(g + 2, p).start()
                in_dur(g + 2, p).start()

            @pl.when(g >= 1)
            def _():
                for c in out_copies(g - 1, pn):
                    c.wait()

            @pl.when(g + 1 < n_chunks)
            def _():
                fire_gathers(pn)

            for c in out_copies(g, p):
                c.start()

        def macro(t, carry):
            sub_body(2 * t, 0)
            sub_body(2 * t + 1, 1)
            return carry

        lax.fori_loop(0, n_chunks // 2, macro, 0)
        for c in out_copies(n_chunks - 1, 1):
            c.wait()

    return k


def kernel(x, table):
    b, l, _ = x.shape
    n = b * l
    idx = x[..., 0].astype(jnp.int32).reshape(n)
    dur = x[..., 1].reshape(n)
    table_pad = jnp.pad(table, ((0, 0), (0, _HIDDEN - table.shape[1])))
    return _make_kernel(b, l)(idx, dur, table_pad)
